# Initial kernel scaffold; baseline (speedup 1.0000x reference)
import functools

import jax
import jax.numpy as jnp
from jax import lax
from jax.experimental import pallas as pl
from jax.experimental.pallas import tpu as pltpu
from jax.experimental.pallas import tpu_sc as plsc

_RES = (16, 32, 64, 128, 256, 512, 1024)
_B = 1048576
_NC, _NS = 2, 16
_NW = _NC * _NS
_PER_W = _B // _NW
_C = 1024
_NCHUNK = _PER_W // _C
_NV = _C // 16
_NROW = _C // 128
_OFFS = (0, 256, 1280, 5376, 21760)
_SMALL_TOT = 87296
_CLIP = jnp.float32(1.0 - 1e-6)
_HI = jnp.int32(-65536)
_HALF = jnp.int32(0x8000)


def _unpack(v):
    f0 = plsc.bitcast(lax.shift_left(v, 16), jnp.float32)
    f1 = plsc.bitcast(lax.bitwise_and(v, _HI), jnp.float32)
    return f0, f1


def _interp_pack(v00, v10, v01, v11, fx, fy):
    a00, b00 = _unpack(v00)
    a10, b10 = _unpack(v10)
    a01, b01 = _unpack(v01)
    a11, b11 = _unpack(v11)
    omx = 1.0 - fx
    omy = 1.0 - fy
    a0 = a00 * omx + a10 * fx
    a1 = a01 * omx + a11 * fx
    b0 = b00 * omx + b10 * fx
    b1 = b01 * omx + b11 * fx
    a = a0 * omy + a1 * fy
    b = b0 * omy + b1 * fy
    pa = plsc.bitcast(a, jnp.int32)
    pb = plsc.bitcast(b, jnp.int32)
    lo = lax.shift_right_logical(pa + _HALF, 16)
    hi = lax.bitwise_and(pb + _HALF, _HI)
    return lax.bitwise_or(lo, hi)


def _cell(xv, yv, r):
    xs = xv * jnp.float32(r - 1)
    ys = yv * jnp.float32(r - 1)
    x0 = xs.astype(jnp.int32)
    y0 = ys.astype(jnp.int32)
    fx = xs - x0.astype(jnp.float32)
    fy = ys - y0.astype(jnp.float32)
    return y0 * r + x0, fx, fy


def _sc_body(x_hbm, y_hbm, tbls_hbm, tbl5_hbm, tbl6_hbm, out_hbm,
             tbl_v, x_v, y_v, out_v, idx_v, g_v, sem):
    cid = lax.axis_index("c")
    sid = lax.axis_index("s")
    wid = sid * _NC + cid
    pltpu.sync_copy(tbls_hbm, tbl_v)
    iota7 = lax.iota(jnp.int32, 16) * 7

    def do_chunk(ci, carry):
        base = wid * _PER_W + ci * _C
        pltpu.sync_copy(x_hbm.at[pl.ds(base, _C)], x_v)
        pltpu.sync_copy(y_hbm.at[pl.ds(base, _C)], y_v)

        def pass1(i, c):
            xv = jnp.minimum(jnp.maximum(x_v[pl.ds(i * 16, 16)], 0.0), _CLIP)
            yv = jnp.minimum(jnp.maximum(y_v[pl.ds(i * 16, 16)], 0.0), _CLIP)
            obase = iota7 + i * (16 * 7)
            for li in range(5):
                r = _RES[li]
                i00, fx, fy = _cell(xv, yv, r)
                o = _OFFS[li]
                v00 = plsc.load_gather(tbl_v, [i00 + o])
                v10 = plsc.load_gather(tbl_v, [i00 + (o + 1)])
                v01 = plsc.load_gather(tbl_v, [i00 + (o + r)])
                v11 = plsc.load_gather(tbl_v, [i00 + (o + r + 1)])
                pk = _interp_pack(v00, v10, v01, v11, fx, fy)
                plsc.store_scatter(out_v, [obase + li], pk)
            row = lax.shift_right_logical(i, 3)
            col = lax.bitwise_and(i, 7) * 16
            for li in (5, 6):
                r = _RES[li]
                i00, _, _ = _cell(xv, yv, r)
                idx_v[li - 5, 0, row, pl.ds(col, 16)] = i00
                idx_v[li - 5, 1, row, pl.ds(col, 16)] = i00 + 1
                idx_v[li - 5, 2, row, pl.ds(col, 16)] = i00 + r
                idx_v[li - 5, 3, row, pl.ds(col, 16)] = i00 + (r + 1)
            return c
        lax.fori_loop(0, _NV, pass1, 0)

        for li, tbl_hbm in ((5, tbl5_hbm), (6, tbl6_hbm)):
            cps = [pltpu.async_copy(tbl_hbm.at[idx_v.at[li - 5, k]],
                                    g_v.at[k], sem) for k in range(4)]
            for cp in cps:
                cp.wait()

            def pass2(i, c, li=li):
                r = _RES[li]
                xv = jnp.minimum(jnp.maximum(x_v[pl.ds(i * 16, 16)], 0.0),
                                 _CLIP)
                yv = jnp.minimum(jnp.maximum(y_v[pl.ds(i * 16, 16)], 0.0),
                                 _CLIP)
                _, fx, fy = _cell(xv, yv, r)
                row = lax.shift_right_logical(i, 3)
                col = lax.bitwise_and(i, 7) * 16
                v00 = g_v[0, row, pl.ds(col, 16)]
                v10 = g_v[1, row, pl.ds(col, 16)]
                v01 = g_v[2, row, pl.ds(col, 16)]
                v11 = g_v[3, row, pl.ds(col, 16)]
                pk = _interp_pack(v00, v10, v01, v11, fx, fy)
                plsc.store_scatter(out_v, [iota7 + (i * (16 * 7) + li)], pk)
                return c
            lax.fori_loop(0, _NV, pass2, 0)

        pltpu.sync_copy(out_v, out_hbm.at[pl.ds(base * 7, _C * 7)])
        return carry
    lax.fori_loop(0, _NCHUNK, do_chunk, 0)


_sc_call = functools.partial(
    pl.kernel,
    out_type=jax.ShapeDtypeStruct((_B * 7,), jnp.int32),
    mesh=plsc.VectorSubcoreMesh(core_axis_name="c", subcore_axis_name="s",
                                num_cores=_NC, num_subcores=_NS),
    scratch_types=[
        pltpu.VMEM((_SMALL_TOT,), jnp.int32),
        pltpu.VMEM((_C,), jnp.float32),
        pltpu.VMEM((_C,), jnp.float32),
        pltpu.VMEM((_C * 7,), jnp.int32),
        pltpu.VMEM((2, 4, _NROW, 128), jnp.int32),
        pltpu.VMEM((4, _NROW, 128), jnp.int32),
        pltpu.SemaphoreType.DMA,
    ],
)(_sc_body)


def kernel(coords, grid0, grid1, grid2, grid3, grid4, grid5, grid6):
    grids = (grid0, grid1, grid2, grid3, grid4, grid5, grid6)
    x = coords[:, 0]
    y = coords[:, 1]
    packed = [lax.bitcast_convert_type(g.astype(jnp.bfloat16), jnp.int32)
              for g in grids]
    small = jnp.concatenate(packed[:5])
    out_i32 = _sc_call(x, y, small, packed[5], packed[6])
    pairs = lax.bitcast_convert_type(out_i32.reshape(_B, 7), jnp.bfloat16)
    return pairs.reshape(_B, 14).astype(jnp.float16)


# speedup vs baseline: 113.3821x; 113.3821x over previous
import functools

import numpy as np
import jax
import jax.numpy as jnp
from jax import lax
from jax.experimental import pallas as pl
from jax.experimental.pallas import tpu as pltpu
from jax.experimental.pallas import tpu_sc as plsc

_RES = (16, 32, 64, 128, 256, 512, 1024)
_B = 1048576
_NC, _NS = 2, 16
_NW = _NC * _NS
_PER_W = _B // _NW
_C = 1024
_NCHUNK = _PER_W // _C
_NV = _C // 16
_NROW = _C // 128
_OFFS = (0, 256, 1280, 5376, 21760)
_SMALL_TOT = 87296
_CLIP = np.float32(1.0 - 1e-6)
_HI = np.int32(-65536)
_HALF = np.int32(0x8000)


def _unpack(v):
    f0 = plsc.bitcast(lax.shift_left(v, 16), jnp.float32)
    f1 = plsc.bitcast(lax.bitwise_and(v, _HI), jnp.float32)
    return f0, f1


def _interp_pack(v00, v10, v01, v11, fx, fy):
    a00, b00 = _unpack(v00)
    a10, b10 = _unpack(v10)
    a01, b01 = _unpack(v01)
    a11, b11 = _unpack(v11)
    omx = 1.0 - fx
    omy = 1.0 - fy
    a0 = a00 * omx + a10 * fx
    a1 = a01 * omx + a11 * fx
    b0 = b00 * omx + b10 * fx
    b1 = b01 * omx + b11 * fx
    a = a0 * omy + a1 * fy
    b = b0 * omy + b1 * fy
    pa = plsc.bitcast(a, jnp.int32)
    pb = plsc.bitcast(b, jnp.int32)
    lo = lax.shift_right_logical(pa + _HALF, 16)
    hi = lax.bitwise_and(pb + _HALF, _HI)
    return lax.bitwise_or(lo, hi)


def _cell(xv, yv, r):
    xs = xv * np.float32(r - 1)
    ys = yv * np.float32(r - 1)
    x0 = xs.astype(jnp.int32)
    y0 = ys.astype(jnp.int32)
    fx = xs - x0.astype(jnp.float32)
    fy = ys - y0.astype(jnp.float32)
    return y0 * r + x0, fx, fy


def _sc_body(x_hbm, y_hbm, tbls_hbm, tbl5_hbm, tbl6_hbm, out_hbm,
             tbl_v, x_v, y_v, out_v,
             i5a, i5b, i5c, i5d, i6a, i6b, i6c, i6d,
             g0, g1, g2, g3, sem):
    cid = lax.axis_index("c")
    sid = lax.axis_index("s")
    wid = sid * _NC + cid
    pltpu.sync_copy(tbls_hbm, tbl_v)
    iota7 = lax.iota(jnp.int32, 16) * 7

    gs = (g0, g1, g2, g3)

    def do_chunk(ci, carry):
        base = wid * _PER_W + ci * _C
        pltpu.sync_copy(x_hbm.at[pl.ds(base, _C)], x_v)
        pltpu.sync_copy(y_hbm.at[pl.ds(base, _C)], y_v)

        def pass1(i, c):
            xv = jnp.minimum(jnp.maximum(x_v[pl.ds(i * 16, 16)], 0.0), _CLIP)
            yv = jnp.minimum(jnp.maximum(y_v[pl.ds(i * 16, 16)], 0.0), _CLIP)
            obase = iota7 + i * (16 * 7)
            for li in range(5):
                r = _RES[li]
                i00, fx, fy = _cell(xv, yv, r)
                o = _OFFS[li]
                v00 = plsc.load_gather(tbl_v, [i00 + o])
                v10 = plsc.load_gather(tbl_v, [i00 + (o + 1)])
                v01 = plsc.load_gather(tbl_v, [i00 + (o + r)])
                v11 = plsc.load_gather(tbl_v, [i00 + (o + r + 1)])
                pk = _interp_pack(v00, v10, v01, v11, fx, fy)
                plsc.store_scatter(out_v, [obase + li], pk)
            for li, (ia, ib, ic, id_) in ((5, (i5a, i5b, i5c, i5d)),
                                          (6, (i6a, i6b, i6c, i6d))):
                r = _RES[li]
                i00, _, _ = _cell(xv, yv, r)
                ia[pl.ds(i * 16, 16)] = i00
                ib[pl.ds(i * 16, 16)] = i00 + 1
                ic[pl.ds(i * 16, 16)] = i00 + r
                id_[pl.ds(i * 16, 16)] = i00 + (r + 1)
            return c
        lax.fori_loop(0, _NV, pass1, 0)

        for li, tbl_hbm, idxs in ((5, tbl5_hbm, (i5a, i5b, i5c, i5d)),
                                  (6, tbl6_hbm, (i6a, i6b, i6c, i6d))):
            cps = [pltpu.async_copy(tbl_hbm.at[idxs[k]], gs[k], sem)
                   for k in range(4)]
            for cp in cps:
                cp.wait()

            def pass2(i, c, li=li):
                r = _RES[li]
                xv = jnp.minimum(jnp.maximum(x_v[pl.ds(i * 16, 16)], 0.0),
                                 _CLIP)
                yv = jnp.minimum(jnp.maximum(y_v[pl.ds(i * 16, 16)], 0.0),
                                 _CLIP)
                _, fx, fy = _cell(xv, yv, r)
                v00 = g0[pl.ds(i * 16, 16)]
                v10 = g1[pl.ds(i * 16, 16)]
                v01 = g2[pl.ds(i * 16, 16)]
                v11 = g3[pl.ds(i * 16, 16)]
                pk = _interp_pack(v00, v10, v01, v11, fx, fy)
                plsc.store_scatter(out_v, [iota7 + (i * (16 * 7) + li)], pk)
                return c
            lax.fori_loop(0, _NV, pass2, 0)

        pltpu.sync_copy(out_v, out_hbm.at[pl.ds(base * 7, _C * 7)])
        return carry
    lax.fori_loop(0, _NCHUNK, do_chunk, 0)


_sc_call = functools.partial(
    pl.kernel,
    out_type=jax.ShapeDtypeStruct((_B * 7,), jnp.int32),
    mesh=plsc.VectorSubcoreMesh(core_axis_name="c", subcore_axis_name="s",
                                num_cores=_NC, num_subcores=_NS),
    scratch_types=[
        pltpu.VMEM((_SMALL_TOT,), jnp.int32),
        pltpu.VMEM((_C,), jnp.float32),
        pltpu.VMEM((_C,), jnp.float32),
        pltpu.VMEM((_C * 7,), jnp.int32),
        pltpu.VMEM((_C,), jnp.int32),
        pltpu.VMEM((_C,), jnp.int32),
        pltpu.VMEM((_C,), jnp.int32),
        pltpu.VMEM((_C,), jnp.int32),
        pltpu.VMEM((_C,), jnp.int32),
        pltpu.VMEM((_C,), jnp.int32),
        pltpu.VMEM((_C,), jnp.int32),
        pltpu.VMEM((_C,), jnp.int32),
        pltpu.VMEM((_C,), jnp.int32),
        pltpu.VMEM((_C,), jnp.int32),
        pltpu.VMEM((_C,), jnp.int32),
        pltpu.VMEM((_C,), jnp.int32),
        pltpu.SemaphoreType.DMA,
    ],
    compiler_params=pltpu.CompilerParams(needs_layout_passes=False),
)(_sc_body)


def kernel(coords, grid0, grid1, grid2, grid3, grid4, grid5, grid6):
    grids = (grid0, grid1, grid2, grid3, grid4, grid5, grid6)
    x = coords[:, 0]
    y = coords[:, 1]
    packed = [lax.bitcast_convert_type(g.astype(jnp.bfloat16), jnp.int32)
              for g in grids]
    small = jnp.concatenate(packed[:5])
    out_i32 = _sc_call(x, y, small, packed[5], packed[6])
    pairs = lax.bitcast_convert_type(out_i32.reshape(_B, 7), jnp.bfloat16)
    return pairs.reshape(_B, 14).astype(jnp.float16)
